# baseline (speedup 1.0000x reference)
import jax
import jax.numpy as jnp
from jax import lax
from jax.experimental import pallas as pl
from jax.experimental.pallas import tpu as pltpu, tpu_sc as plsc

NUM_CLASSES = 1000
NUM_GROUPS = 10
BATCH = 16384
COST = 0.05
EPS = 1e-12

_L = 16
_NVEC = 63
_TAIL = NUM_CLASSES - _L
_KUN = 3

_info = plsc.get_sparse_core_info()
_NC, _NS = _info.num_cores, _info.num_subcores
_NW = _NC * _NS
_CHUNK = 16
_NBUF = 6
_BIG = 1 << 30


def _sc_body(rows_per_w, post_hbm, c2g_hbm, alpha_hbm, mu_hbm,
             pred_hbm, rej_hbm, *refs):
    nch = rows_per_w // _CHUNK
    bufs = refs[:_NBUF]
    ta, tw, c2gv, av, mv, pm, pi, pa, po, ro = refs[_NBUF:_NBUF + 10]
    sems = refs[_NBUF + 10:]
    wid = lax.axis_index("s") * _NC + lax.axis_index("c")
    base_row = wid * rows_per_w

    def chunk_copy(ci, b):
        return pltpu.make_async_copy(
            post_hbm.at[pl.ds(base_row + ci * _CHUNK, _CHUNK), :],
            bufs[b], sems[b])

    for b in range(_NBUF):
        chunk_copy(b, b).start()

    iota = lax.iota(jnp.int32, _L)
    iota16 = iota * _L

    pltpu.sync_copy(c2g_hbm, c2gv)
    for o in range(0, NUM_CLASSES, 128):
        n = min(128, NUM_CLASSES - o)
        isl = pl.ds(o, n)
        pltpu.sync_copy(alpha_hbm.at[c2gv.at[isl]], av.at[isl])
        pltpu.sync_copy(mu_hbm.at[c2gv.at[isl]], mv.at[isl])

    for k in range(_NVEC):
        cb = _L * k if k < _NVEC - 1 else _TAIL
        sl0 = pl.ds(cb, _L)
        ah = jnp.maximum(av[sl0] / float(NUM_GROUPS), EPS)
        w = 1.0 / ah - mv[sl0]
        if k == _NVEC - 1:
            w = jnp.where(iota < 8, 0.0, w)
        sl = pl.ds(_L * k, _L)
        ta[sl] = ah
        tw[sl] = w

    def slice8(buf, h, k, carry):
        ms, idxs, accs = carry
        ko = k * _L
        o = jnp.where(k < _NVEC - 1, ko, _TAIL)
        idxv = o + iota
        tsl = pl.ds(ko, _L)
        tav = ta[tsl]
        twv = tw[tsl]
        ms2, idxs2, accs2 = [], [], []
        for r in range(8):
            p = buf[h * 8 + r, pl.ds(o, _L)]
            q = p / tav
            upd = q > ms[r]
            ms2.append(jnp.maximum(ms[r], q))
            idxs2.append(jnp.where(upd, idxv, idxs[r]))
            accs2.append(accs[r] + twv * p)
        return tuple(ms2), tuple(idxs2), tuple(accs2)

    def compute_chunk(buf, out0):
        def h_body(h, _):
            init = (tuple(jnp.full((_L,), -1.0, jnp.float32)
                          for _ in range(8)),
                    tuple(jnp.zeros((_L,), jnp.int32) for _ in range(8)),
                    tuple(jnp.zeros((_L,), jnp.float32) for _ in range(8)))

            def kbody(i, carry):
                for t in range(_KUN):
                    carry = slice8(buf, h, i * _KUN + t, carry)
                return carry

            ms, idxs, accs = lax.fori_loop(0, _NVEC // _KUN, kbody, init)
            for r in range(8):
                psl = pl.ds(h * 8 * _L + r * _L, _L)
                pm[psl] = ms[r]
                pi[psl] = idxs[r]
                pa[psl] = accs[r]
            return 0

        lax.fori_loop(0, 2, h_body, 0)

        pmf, pif, paf = pm, pi, pa
        vm = [plsc.load_gather(pmf, [iota16 + j]) for j in range(_L)]
        mx = vm[0]
        for j in range(1, _L):
            mx = jnp.maximum(mx, vm[j])
        vi = [plsc.load_gather(pif, [iota16 + j]) for j in range(_L)]
        pred = jnp.full((_L,), _BIG, jnp.int32)
        for j in range(_L):
            pred = jnp.minimum(pred, jnp.where(vm[j] == mx, vi[j], _BIG))
        va = [plsc.load_gather(paf, [iota16 + j]) for j in range(_L)]
        thr = va[0]
        for j in range(1, _L):
            thr = thr + va[j]
        rj = jnp.where(mx < thr - COST, 1, 0)
        osl = pl.ds(out0, _L)
        po[osl] = pred
        ro[osl] = rj

    def ring_body(g, _):
        for b in range(_NBUF):
            ci = g * _NBUF + b

            @pl.when(ci < nch)
            def _():
                chunk_copy(ci, b).wait()
                compute_chunk(bufs[b], ci * _CHUNK)

            @pl.when(ci + _NBUF < nch)
            def _():
                chunk_copy(ci + _NBUF, b).start()
        return 0

    lax.fori_loop(0, (nch + _NBUF - 1) // _NBUF, ring_body, 0)
    pltpu.sync_copy(po, pred_hbm.at[pl.ds(base_row, rows_per_w)])
    pltpu.sync_copy(ro, rej_hbm.at[pl.ds(base_row, rows_per_w)])


def _make_sc_call(batch):
    rows_per_w = batch // _NW

    def body(*args):
        return _sc_body(rows_per_w, *args)

    return pl.kernel(
        body,
        out_type=[jax.ShapeDtypeStruct((batch,), jnp.int32),
                  jax.ShapeDtypeStruct((batch,), jnp.int32)],
        mesh=plsc.VectorSubcoreMesh(core_axis_name="c",
                                    subcore_axis_name="s"),
        compiler_params=pltpu.CompilerParams(needs_layout_passes=False,
                                             use_tc_tiling_on_sc=True),
        scratch_types=(
            [pltpu.VMEM((_CHUNK, NUM_CLASSES), jnp.float32)
             for _ in range(_NBUF)]
            + [pltpu.VMEM((_NVEC * _L,), jnp.float32),
               pltpu.VMEM((_NVEC * _L,), jnp.float32),
               pltpu.VMEM((NUM_CLASSES,), jnp.int32),
               pltpu.VMEM((NUM_CLASSES,), jnp.float32),
               pltpu.VMEM((NUM_CLASSES,), jnp.float32),
               pltpu.VMEM((2 * 8 * _L,), jnp.float32),
               pltpu.VMEM((2 * 8 * _L,), jnp.int32),
               pltpu.VMEM((2 * 8 * _L,), jnp.float32),
               pltpu.VMEM((rows_per_w,), jnp.int32),
               pltpu.VMEM((rows_per_w,), jnp.int32)]
            + [pltpu.SemaphoreType.DMA for _ in range(_NBUF)]
        ),
    )


_sc_call = _make_sc_call(BATCH)


@jax.jit
def kernel(posterior, class_to_group, alpha_group, mu_group):
    pad = 128 - NUM_GROUPS
    pred, rej = _sc_call(posterior, class_to_group,
                         jnp.pad(alpha_group, (0, pad), constant_values=1.0),
                         jnp.pad(mu_group, (0, pad)))
    return pred, rej.astype(jnp.bool_)
